# Initial kernel scaffold; baseline (speedup 1.0000x reference)
import jax
import jax.numpy as jnp
from jax.experimental import pallas as pl


def kernel(x, emb_table, W1, W2):
    raise NotImplementedError("write your pallas kernel here")


# speedup vs baseline: 2.0811x; 2.0811x over previous
import functools

import jax
import jax.numpy as jnp
import numpy as np
from jax import lax
from jax.experimental import pallas as pl
from jax.experimental.pallas import tpu as pltpu
from jax.experimental.pallas import tpu_sc as plsc

B = 4096
L = 50
VOCAB = 100000
D = 128

NC = 2
NS = 16
NW = NC * NS
ROWS = B * L
RPW = ROWS // NW
CHUNK = 200
GSUB = 104
GSUB2 = CHUNK - GSUB
NCHUNK = RPW // CHUNK
BPW = B // NW
NSLICE = D // 16


def _set_max(rows_v, base_r):
    accs = tuple(rows_v[base_r, pl.ds(k * 16, 16)] for k in range(NSLICE))

    def row_body(r, accs):
        return tuple(
            jnp.maximum(a, rows_v[r, pl.ds(k * 16, 16)])
            for k, a in enumerate(accs)
        )

    return lax.fori_loop(base_r + 1, base_r + L, row_body, accs)


def _sc_gather_body(table_hbm, idx_hbm, e_hbm, maxp_hbm,
                    idx_v, rows_v, maxs_v, sem):
    wid = lax.axis_index("s") * NC + lax.axis_index("c")
    pltpu.sync_copy(idx_hbm.at[pl.ds(wid * RPW, RPW)], idx_v)

    def chunk_body(j, carry):
        pltpu.async_copy(table_hbm.at[idx_v.at[pl.ds(j * CHUNK, GSUB)]],
                         rows_v.at[pl.ds(0, GSUB)], sem)
        pltpu.async_copy(table_hbm.at[idx_v.at[pl.ds(j * CHUNK + GSUB, GSUB2)]],
                         rows_v.at[pl.ds(GSUB, GSUB2)], sem).wait()
        pltpu.make_async_copy(table_hbm.at[idx_v.at[pl.ds(j * CHUNK, GSUB)]],
                              rows_v.at[pl.ds(0, GSUB)], sem).wait()
        pltpu.sync_copy(rows_v, e_hbm.at[pl.ds(wid * RPW + j * CHUNK, CHUNK)])
        for g in range(CHUNK // L):
            accs = _set_max(rows_v, g * L)
            for k in range(NSLICE):
                maxs_v[j * (CHUNK // L) + g, pl.ds(k * 16, 16)] = accs[k]
        return carry

    lax.fori_loop(0, NCHUNK, chunk_body, 0)
    pltpu.sync_copy(maxs_v, maxp_hbm.at[pl.ds(wid * BPW, BPW)])


def _sc_gather(emb_table, idx_flat):
    return pl.kernel(
        _sc_gather_body,
        out_type=(
            jax.ShapeDtypeStruct((ROWS, D), jnp.float32),
            jax.ShapeDtypeStruct((B, D), jnp.float32),
        ),
        mesh=plsc.VectorSubcoreMesh(core_axis_name="c", subcore_axis_name="s"),
        scratch_types=[
            pltpu.VMEM((RPW,), jnp.int32),
            pltpu.VMEM((CHUNK, D), jnp.float32),
            pltpu.VMEM((BPW, D), jnp.float32),
            pltpu.SemaphoreType.DMA,
        ],
    )(emb_table, idx_flat)


BT = 32

_SEL_NP = np.zeros((BT * L, BT), np.float32)
_SEL_NP[np.arange(BT * L), np.arange(BT * L) // L] = 1.0


def _tc_body(e_ref, mp_ref, w1_ref, w2_ref, s_ref, o_ref):
    cdims = (((1,), (1,)), ((), ()))
    m2 = lax.dot_general(mp_ref[...], w2_ref[...], cdims,
                         preferred_element_type=jnp.float32)
    e1 = lax.dot_general(e_ref[...], w1_ref[...], cdims,
                         preferred_element_type=jnp.float32)
    bcast = lax.dot_general(s_ref[...], m2, (((1,), (0,)), ((), ())),
                            preferred_element_type=jnp.float32)
    c = e1 + bcast
    o_ref[...] = 0.5 * jnp.tanh(0.5 * c) + 0.5


def _tc_fused(e, maxp, W1, W2):
    grid = B // BT
    return pl.pallas_call(
        _tc_body,
        grid=(grid,),
        in_specs=[
            pl.BlockSpec((BT * L, D), lambda i: (i, 0)),
            pl.BlockSpec((BT, D), lambda i: (i, 0)),
            pl.BlockSpec((D, D), lambda i: (0, 0)),
            pl.BlockSpec((D, D), lambda i: (0, 0)),
            pl.BlockSpec((BT * L, BT), lambda i: (0, 0)),
        ],
        out_specs=pl.BlockSpec((BT * L, D), lambda i: (i, 0)),
        out_shape=jax.ShapeDtypeStruct((ROWS, D), jnp.float32),
    )(e, maxp, W1, W2, jnp.asarray(_SEL_NP))


def kernel(x, emb_table, W1, W2):
    idx_flat = x.reshape(-1).astype(jnp.int32)
    e, maxp = _sc_gather(emb_table, idx_flat)
    out = _tc_fused(e, maxp, W1, W2)
    return out.reshape(B, L, D)


# speedup vs baseline: 2.7483x; 1.3206x over previous
import functools

import jax
import jax.numpy as jnp
import numpy as np
from jax import lax
from jax.experimental import pallas as pl
from jax.experimental.pallas import tpu as pltpu
from jax.experimental.pallas import tpu_sc as plsc

B = 4096
L = 50
VOCAB = 100000
D = 128

NC = 2
NS = 16
NW = NC * NS
ROWS = B * L
RPW = ROWS // NW
CHUNK = 200
GSUB = 104
GSUB2 = CHUNK - GSUB
NCHUNK = RPW // CHUNK
BPW = B // NW
NSLICE = D // 16


def _set_max(rows_v, base_r):
    accs = tuple(rows_v[base_r, pl.ds(k * 16, 16)] for k in range(NSLICE))

    def row_body(r, accs):
        return tuple(
            jnp.maximum(a, rows_v[r, pl.ds(k * 16, 16)])
            for k, a in enumerate(accs)
        )

    return lax.fori_loop(base_r + 1, base_r + L, row_body, accs)


def _sc_gather_body(table_hbm, idx_hbm, e_hbm, maxp_hbm,
                    idx_v, rows_v, maxs_v, sem):
    wid = lax.axis_index("s") * NC + lax.axis_index("c")
    pltpu.sync_copy(idx_hbm.at[pl.ds(wid * RPW, RPW)], idx_v)

    def chunk_body(j, carry):
        pltpu.async_copy(table_hbm.at[idx_v.at[pl.ds(j * CHUNK, GSUB)]],
                         rows_v.at[pl.ds(0, GSUB)], sem)
        pltpu.async_copy(table_hbm.at[idx_v.at[pl.ds(j * CHUNK + GSUB, GSUB2)]],
                         rows_v.at[pl.ds(GSUB, GSUB2)], sem).wait()
        pltpu.make_async_copy(table_hbm.at[idx_v.at[pl.ds(j * CHUNK, GSUB)]],
                              rows_v.at[pl.ds(0, GSUB)], sem).wait()
        pltpu.sync_copy(rows_v, e_hbm.at[pl.ds(wid * RPW + j * CHUNK, CHUNK)])
        for g in range(CHUNK // L):
            accs = _set_max(rows_v, g * L)
            for k in range(NSLICE):
                maxs_v[j * (CHUNK // L) + g, pl.ds(k * 16, 16)] = accs[k]
        return carry

    lax.fori_loop(0, NCHUNK, chunk_body, 0)
    pltpu.sync_copy(maxs_v, maxp_hbm.at[pl.ds(wid * BPW, BPW)])


def _sc_gather(emb_table, idx_flat):
    return pl.kernel(
        _sc_gather_body,
        out_type=(
            jax.ShapeDtypeStruct((ROWS, D), jnp.float32),
            jax.ShapeDtypeStruct((B, D), jnp.float32),
        ),
        mesh=plsc.VectorSubcoreMesh(core_axis_name="c", subcore_axis_name="s"),
        scratch_types=[
            pltpu.VMEM((RPW,), jnp.int32),
            pltpu.VMEM((CHUNK, D), jnp.float32),
            pltpu.VMEM((BPW, D), jnp.float32),
            pltpu.SemaphoreType.DMA,
        ],
    )(emb_table, idx_flat)


BT = 32

_SEL_NP = np.zeros((BT * L, BT), np.float32)
_SEL_NP[np.arange(BT * L), np.arange(BT * L) // L] = 1.0


def _tc_body(e_ref, mp_ref, w1_ref, w2_ref, s_ref, o_ref):
    cdims = (((1,), (1,)), ((), ()))
    m2 = lax.dot_general(mp_ref[...], w2_ref[...], cdims,
                         preferred_element_type=jnp.float32)
    e1 = lax.dot_general(e_ref[...], w1_ref[...], cdims,
                         preferred_element_type=jnp.float32)
    bcast = lax.dot_general(s_ref[...], m2, (((1,), (0,)), ((), ())),
                            preferred_element_type=jnp.float32)
    c = e1 + bcast
    o_ref[...] = (0.5 * jnp.tanh(0.5 * c) + 0.5).reshape(BT, L, D)


def _tc_fused(e, maxp, W1, W2):
    grid = B // BT
    return pl.pallas_call(
        _tc_body,
        grid=(grid,),
        in_specs=[
            pl.BlockSpec((BT * L, D), lambda i: (i, 0)),
            pl.BlockSpec((BT, D), lambda i: (i, 0)),
            pl.BlockSpec((D, D), lambda i: (0, 0)),
            pl.BlockSpec((D, D), lambda i: (0, 0)),
            pl.BlockSpec((BT * L, BT), lambda i: (0, 0)),
        ],
        out_specs=pl.BlockSpec((BT, L, D), lambda i: (i, 0, 0)),
        out_shape=jax.ShapeDtypeStruct((B, L, D), jnp.float32),
    )(e, maxp, W1, W2, jnp.asarray(_SEL_NP))


def kernel(x, emb_table, W1, W2):
    idx_flat = x.reshape(-1).astype(jnp.int32)
    e, maxp = _sc_gather(emb_table, idx_flat)
    return _tc_fused(e, maxp, W1, W2)


# speedup vs baseline: 3.7982x; 1.3820x over previous
import functools

import jax
import jax.numpy as jnp
import numpy as np
from jax import lax
from jax.experimental import pallas as pl
from jax.experimental.pallas import tpu as pltpu
from jax.experimental.pallas import tpu_sc as plsc

B = 4096
L = 50
VOCAB = 100000
D = 128

NC = 2
NS = 16
NW = NC * NS
ROWS = B * L
RPW = ROWS // NW
SETS = 8
CHUNK = SETS * L
NCHUNK = RPW // CHUNK
BPW = B // NW
NSLICE = D // 16
GOFF = (0, 104, 208, 312)
GLEN = (104, 104, 104, 88)


def _sc_gather_body(table_hbm, idx_hbm, e_hbm, maxp_hbm,
                    idx_v, rows_v, maxs_v, semg, semw):
    wid = lax.axis_index("s") * NC + lax.axis_index("c")
    pltpu.sync_copy(idx_hbm.at[pl.ds(wid * RPW, RPW)], idx_v)

    def fire_gathers(j):
        for o, n in zip(GOFF, GLEN):
            pltpu.async_copy(table_hbm.at[idx_v.at[pl.ds(j * CHUNK + o, n)]],
                             rows_v.at[pl.ds(o, n)], semg)

    def wait_gathers(j):
        for o, n in zip(GOFF, GLEN):
            pltpu.make_async_copy(
                table_hbm.at[idx_v.at[pl.ds(j * CHUNK + o, n)]],
                rows_v.at[pl.ds(o, n)], semg).wait()

    def chunk_body(j, carry):
        fire_gathers(j)
        wait_gathers(j)
        gb = wid * BPW + j * SETS

        def wb_fire(l, c):
            pltpu.async_copy(rows_v.at[pl.ds(l * SETS, SETS)],
                             e_hbm.at[l, pl.ds(gb, SETS)], semw)
            return c

        lax.fori_loop(0, L, wb_fire, 0)

        for b in range(SETS):
            accs = tuple(rows_v[b, pl.ds(k * 16, 16)] for k in range(NSLICE))

            def row_body(l, accs):
                return tuple(
                    jnp.maximum(a, rows_v[l * SETS + b, pl.ds(k * 16, 16)])
                    for k, a in enumerate(accs)
                )

            accs = lax.fori_loop(1, L, row_body, accs)
            for k in range(NSLICE):
                maxs_v[j * SETS + b, pl.ds(k * 16, 16)] = accs[k]

        def wb_drain(l, c):
            pltpu.make_async_copy(rows_v.at[pl.ds(l * SETS, SETS)],
                                  e_hbm.at[l, pl.ds(gb, SETS)], semw).wait()
            return c

        lax.fori_loop(0, L, wb_drain, 0)
        return carry

    lax.fori_loop(0, NCHUNK, chunk_body, 0)
    pltpu.sync_copy(maxs_v, maxp_hbm.at[pl.ds(wid * BPW, BPW)])


def _sc_gather(emb_table, idx_flat):
    return pl.kernel(
        _sc_gather_body,
        out_type=(
            jax.ShapeDtypeStruct((L, B, D), jnp.float32),
            jax.ShapeDtypeStruct((B, D), jnp.float32),
        ),
        mesh=plsc.VectorSubcoreMesh(core_axis_name="c", subcore_axis_name="s"),
        scratch_types=[
            pltpu.VMEM((RPW,), jnp.int32),
            pltpu.VMEM((CHUNK, D), jnp.float32),
            pltpu.VMEM((BPW, D), jnp.float32),
            pltpu.SemaphoreType.DMA,
            pltpu.SemaphoreType.DMA,
        ],
    )(emb_table, idx_flat)


BT = 32

_SEL_NP = np.zeros((L * BT, BT), np.float32)
_SEL_NP[np.arange(L * BT), np.arange(L * BT) % BT] = 1.0


def _tc_body(e_ref, mp_ref, w1_ref, w2_ref, s_ref, o_ref):
    cdims = (((1,), (1,)), ((), ()))
    e = e_ref[...].reshape(L * BT, D)
    m2 = lax.dot_general(mp_ref[...], w2_ref[...], cdims,
                         preferred_element_type=jnp.float32)
    e1 = lax.dot_general(e, w1_ref[...], cdims,
                         preferred_element_type=jnp.float32)
    bcast = lax.dot_general(s_ref[...], m2, (((1,), (0,)), ((), ())),
                            preferred_element_type=jnp.float32)
    c = e1 + bcast
    o_ref[...] = (0.5 * jnp.tanh(0.5 * c) + 0.5).reshape(L, BT, D)


def _tc_fused(e, maxp, W1, W2):
    grid = B // BT
    return pl.pallas_call(
        _tc_body,
        grid=(grid,),
        in_specs=[
            pl.BlockSpec((L, BT, D), lambda i: (0, i, 0)),
            pl.BlockSpec((BT, D), lambda i: (i, 0)),
            pl.BlockSpec((D, D), lambda i: (0, 0)),
            pl.BlockSpec((D, D), lambda i: (0, 0)),
            pl.BlockSpec((L * BT, BT), lambda i: (0, 0)),
        ],
        out_specs=pl.BlockSpec((L, BT, D), lambda i: (0, i, 0)),
        out_shape=jax.ShapeDtypeStruct((L, B, D), jnp.float32),
    )(e, maxp, W1, W2, jnp.asarray(_SEL_NP))


def kernel(x, emb_table, W1, W2):
    idx_flat = (x.astype(jnp.int32)
                 .reshape(B // SETS, SETS, L)
                 .transpose(0, 2, 1)
                 .reshape(-1))
    e, maxp = _sc_gather(emb_table, idx_flat)
    out_t = _tc_fused(e, maxp, W1, W2)
    return jnp.transpose(out_t, (1, 0, 2))


# speedup vs baseline: 4.1086x; 1.0817x over previous
import functools

import jax
import jax.numpy as jnp
import numpy as np
from jax import lax
from jax.experimental import pallas as pl
from jax.experimental.pallas import tpu as pltpu
from jax.experimental.pallas import tpu_sc as plsc

B = 4096
L = 50
VOCAB = 100000
D = 128

NC = 2
NS = 16
NW = NC * NS
BPW = B // NW


def _sc_gather_body(table_hbm, xt_hbm, e_hbm,
                    slab_v, rows_a, rows_b, sga, sgb, swa, swb):
    wid = lax.axis_index("s") * NC + lax.axis_index("c")
    base = wid * BPW
    pltpu.sync_copy(xt_hbm.at[:, pl.ds(base, BPW)], slab_v)

    def fire_g(l, buf, sem):
        pltpu.async_copy(table_hbm.at[slab_v.at[l]], buf, sem)

    def wait_g(l, buf, sem):
        pltpu.make_async_copy(table_hbm.at[slab_v.at[l]], buf, sem).wait()

    def fire_w(l, buf, sem):
        pltpu.async_copy(buf, e_hbm.at[l, pl.ds(base, BPW)], sem)

    def wait_w(l, buf, sem):
        pltpu.make_async_copy(buf, e_hbm.at[l, pl.ds(base, BPW)], sem).wait()

    fire_g(0, rows_a, sga)
    fire_g(1, rows_b, sgb)

    def pair(i, carry):
        la = 2 * i
        lb = 2 * i + 1
        wait_g(la, rows_a, sga)
        fire_w(la, rows_a, swa)
        wait_g(lb, rows_b, sgb)
        fire_w(lb, rows_b, swb)
        wait_w(la, rows_a, swa)

        @pl.when(la + 2 < L)
        def _():
            fire_g(la + 2, rows_a, sga)

        wait_w(lb, rows_b, swb)

        @pl.when(lb + 2 < L)
        def _():
            fire_g(lb + 2, rows_b, sgb)

        return carry

    lax.fori_loop(0, L // 2, pair, 0)


def _sc_gather(emb_table, xt):
    return pl.kernel(
        _sc_gather_body,
        out_type=jax.ShapeDtypeStruct((L, B, D), jnp.float32),
        mesh=plsc.VectorSubcoreMesh(core_axis_name="c", subcore_axis_name="s"),
        scratch_types=[
            pltpu.VMEM((L, BPW), jnp.int32),
            pltpu.VMEM((BPW, D), jnp.float32),
            pltpu.VMEM((BPW, D), jnp.float32),
            pltpu.SemaphoreType.DMA,
            pltpu.SemaphoreType.DMA,
            pltpu.SemaphoreType.DMA,
            pltpu.SemaphoreType.DMA,
        ],
    )(emb_table, xt)


BT = 32

_SEL_NP = np.zeros((L * BT, BT), np.float32)
_SEL_NP[np.arange(L * BT), np.arange(L * BT) % BT] = 1.0


def _tc_body(e_ref, w1_ref, w2_ref, s_ref, o_ref):
    cdims = (((1,), (1,)), ((), ()))
    e3 = e_ref[...]
    m = jnp.max(e3, axis=0)
    e = e3.reshape(L * BT, D)
    m2 = lax.dot_general(m, w2_ref[...], cdims,
                         preferred_element_type=jnp.float32)
    e1 = lax.dot_general(e, w1_ref[...], cdims,
                         preferred_element_type=jnp.float32)
    bcast = lax.dot_general(s_ref[...], m2, (((1,), (0,)), ((), ())),
                            preferred_element_type=jnp.float32)
    c = e1 + bcast
    o_ref[...] = (0.5 * jnp.tanh(0.5 * c) + 0.5).reshape(L, BT, D)


def _tc_fused(e, W1, W2):
    grid = B // BT
    return pl.pallas_call(
        _tc_body,
        grid=(grid,),
        in_specs=[
            pl.BlockSpec((L, BT, D), lambda i: (0, i, 0)),
            pl.BlockSpec((D, D), lambda i: (0, 0)),
            pl.BlockSpec((D, D), lambda i: (0, 0)),
            pl.BlockSpec((L * BT, BT), lambda i: (0, 0)),
        ],
        out_specs=pl.BlockSpec((L, BT, D), lambda i: (0, i, 0)),
        out_shape=jax.ShapeDtypeStruct((L, B, D), jnp.float32),
    )(e, W1, W2, jnp.asarray(_SEL_NP))


def kernel(x, emb_table, W1, W2):
    xt = jnp.transpose(x.astype(jnp.int32), (1, 0))
    e = _sc_gather(emb_table, xt)
    out_t = _tc_fused(e, W1, W2)
    return jnp.transpose(out_t, (1, 0, 2))


# speedup vs baseline: 5.2387x; 1.2751x over previous
import functools

import jax
import jax.numpy as jnp
import numpy as np
from jax import lax
from jax.experimental import pallas as pl
from jax.experimental.pallas import tpu as pltpu
from jax.experimental.pallas import tpu_sc as plsc

B = 4096
L = 50
VOCAB = 100000
D = 128

NC = 2
NS = 16
NW = NC * NS
BPW = B // NW


def _sc_gather_body(table_hbm, xt_hbm, e_hbm,
                    slab_v, rows_a, rows_b, sga, sgb, swa, swb):
    wid = lax.axis_index("s") * NC + lax.axis_index("c")
    base = wid * BPW
    pltpu.sync_copy(xt_hbm.at[:, pl.ds(base, BPW)], slab_v)

    def fire_g(l, buf, sem):
        pltpu.async_copy(table_hbm.at[slab_v.at[l]], buf, sem)

    def wait_g(l, buf, sem):
        pltpu.make_async_copy(table_hbm.at[slab_v.at[l]], buf, sem).wait()

    def fire_w(l, buf, sem):
        pltpu.async_copy(buf, e_hbm.at[l, pl.ds(base, BPW)], sem)

    def wait_w(l, buf, sem):
        pltpu.make_async_copy(buf, e_hbm.at[l, pl.ds(base, BPW)], sem).wait()

    fire_g(0, rows_a, sga)
    fire_g(1, rows_b, sgb)

    def pair(i, carry):
        la = 2 * i
        lb = 2 * i + 1
        wait_g(la, rows_a, sga)
        fire_w(la, rows_a, swa)
        wait_g(lb, rows_b, sgb)
        fire_w(lb, rows_b, swb)
        wait_w(la, rows_a, swa)

        @pl.when(la + 2 < L)
        def _():
            fire_g(la + 2, rows_a, sga)

        wait_w(lb, rows_b, swb)

        @pl.when(lb + 2 < L)
        def _():
            fire_g(lb + 2, rows_b, sgb)

        return carry

    lax.fori_loop(0, L // 2, pair, 0)


def _sc_gather(emb_table, xt):
    return pl.kernel(
        _sc_gather_body,
        out_type=jax.ShapeDtypeStruct((L, B, D), jnp.float32),
        mesh=plsc.VectorSubcoreMesh(core_axis_name="c", subcore_axis_name="s"),
        scratch_types=[
            pltpu.VMEM((L, BPW), jnp.int32),
            pltpu.VMEM((BPW, D), jnp.float32),
            pltpu.VMEM((BPW, D), jnp.float32),
            pltpu.SemaphoreType.DMA,
            pltpu.SemaphoreType.DMA,
            pltpu.SemaphoreType.DMA,
            pltpu.SemaphoreType.DMA,
        ],
    )(emb_table, xt)


BT = 64


def _tc_body(e_ref, w1_ref, w2_ref, o_ref):
    cdims = (((1,), (1,)), ((), ()))
    e3 = e_ref[...]
    m = jnp.max(e3, axis=0)
    e = e3.reshape(L * BT, D)
    m2 = lax.dot_general(m, w2_ref[...], cdims,
                         preferred_element_type=jnp.float32)
    e1 = lax.dot_general(e, w1_ref[...], cdims,
                         preferred_element_type=jnp.float32)
    c = e1.reshape(L, BT, D) + m2[None]
    o_ref[...] = 0.5 * jnp.tanh(0.5 * c) + 0.5


def _tc_fused(e, W1, W2):
    grid = B // BT
    return pl.pallas_call(
        _tc_body,
        grid=(grid,),
        in_specs=[
            pl.BlockSpec((L, BT, D), lambda i: (0, i, 0)),
            pl.BlockSpec((D, D), lambda i: (0, 0)),
            pl.BlockSpec((D, D), lambda i: (0, 0)),
        ],
        out_specs=pl.BlockSpec((L, BT, D), lambda i: (0, i, 0)),
        out_shape=jax.ShapeDtypeStruct((L, B, D), jnp.float32),
    )(e, W1, W2)


def kernel(x, emb_table, W1, W2):
    xt = jnp.transpose(x.astype(jnp.int32), (1, 0))
    e = _sc_gather(emb_table, xt)
    out_t = _tc_fused(e, W1, W2)
    return jnp.transpose(out_t, (1, 0, 2))


# speedup vs baseline: 5.5830x; 1.0657x over previous
import functools

import jax
import jax.numpy as jnp
import numpy as np
from jax import lax
from jax.experimental import pallas as pl
from jax.experimental.pallas import tpu as pltpu
from jax.experimental.pallas import tpu_sc as plsc

B = 4096
L = 50
VOCAB = 100000
D = 128

NC = 2
NS = 16
NW = NC * NS
NQ = 4
QSETS = B // NQ
BPW = QSETS // NW


SLAB = 128
NA = QSETS // SLAB
NB = NW // NA
LQ = 13


def _sc_gather_body(qoff, table_hbm, xt_hbm, e_hbm,
                    slab_v, rows_a, rows_b, sga, sgb, swa, swb):
    wid = lax.axis_index("s") * NC + lax.axis_index("c")
    aw = lax.rem(wid, NA)
    bw = wid // NA
    setbase = aw * SLAB
    lbase = bw * LQ
    lim = jnp.minimum(lbase + LQ, L)
    pltpu.sync_copy(xt_hbm.at[:, pl.ds(qoff + setbase, SLAB)], slab_v)

    def fire_g(l, buf, sem):
        @pl.when(l < lim)
        def _():
            pltpu.async_copy(table_hbm.at[slab_v.at[l]], buf, sem)

    def wait_g(l, buf, sem):
        @pl.when(l < lim)
        def _():
            pltpu.make_async_copy(table_hbm.at[slab_v.at[l]], buf, sem).wait()

    def fire_w(l, buf, sem):
        @pl.when(l < lim)
        def _():
            pltpu.async_copy(buf, e_hbm.at[l, pl.ds(setbase, SLAB)], sem)

    def wait_w(l, buf, sem):
        @pl.when(l < lim)
        def _():
            pltpu.make_async_copy(buf,
                                  e_hbm.at[l, pl.ds(setbase, SLAB)],
                                  sem).wait()

    fire_g(lbase, rows_a, sga)
    fire_g(lbase + 1, rows_b, sgb)

    def pair(i, carry):
        la = lbase + 2 * i
        lb = la + 1
        wait_g(la, rows_a, sga)
        fire_w(la, rows_a, swa)
        wait_g(lb, rows_b, sgb)
        fire_w(lb, rows_b, swb)
        wait_w(la, rows_a, swa)
        fire_g(la + 2, rows_a, sga)
        wait_w(lb, rows_b, swb)
        fire_g(lb + 2, rows_b, sgb)
        return carry

    lax.fori_loop(0, LQ // 2, pair, 0)
    lt = lbase + LQ - 1
    wait_g(lt, rows_a, sga)
    fire_w(lt, rows_a, swa)
    wait_w(lt, rows_a, swa)


def _sc_gather(emb_table, xt, q):
    return pl.kernel(
        functools.partial(_sc_gather_body, q * QSETS),
        out_type=jax.ShapeDtypeStruct((L, QSETS, D), jnp.float32),
        mesh=plsc.VectorSubcoreMesh(core_axis_name="c", subcore_axis_name="s"),
        scratch_types=[
            pltpu.VMEM((L, SLAB), jnp.int32),
            pltpu.VMEM((SLAB, D), jnp.float32),
            pltpu.VMEM((SLAB, D), jnp.float32),
            pltpu.SemaphoreType.DMA,
            pltpu.SemaphoreType.DMA,
            pltpu.SemaphoreType.DMA,
            pltpu.SemaphoreType.DMA,
        ],
    )(emb_table, xt)


BT = 64
QBLOCKS = QSETS // BT


def _tc_compute(e_ref, w1_ref, w2_ref, o_ref):
    cdims = (((1,), (1,)), ((), ()))
    e3 = e_ref[...]
    m = jnp.max(e3, axis=0)
    e = e3.reshape(L * BT, D)
    m2 = lax.dot_general(m, w2_ref[...], cdims,
                         preferred_element_type=jnp.float32)
    e1 = lax.dot_general(e, w1_ref[...], cdims,
                         preferred_element_type=jnp.float32)
    c = e1.reshape(L, BT, D) + m2[None]
    o_ref[...] = 0.5 * jnp.tanh(0.5 * c) + 0.5


def _tc_body_first(e_ref, w1_ref, w2_ref, o_ref):
    _tc_compute(e_ref, w1_ref, w2_ref, o_ref)


def _tc_body_next(e_ref, w1_ref, w2_ref, o_prev_ref, o_ref):
    del o_prev_ref
    _tc_compute(e_ref, w1_ref, w2_ref, o_ref)


def _tc_stripe(e_q, W1, W2, q, o_prev):
    out_spec = pl.BlockSpec((L, BT, D), lambda i, q=q: (0, q * QBLOCKS + i, 0))
    in_specs = [
        pl.BlockSpec((L, BT, D), lambda i: (0, i, 0)),
        pl.BlockSpec((D, D), lambda i: (0, 0)),
        pl.BlockSpec((D, D), lambda i: (0, 0)),
    ]
    if o_prev is None:
        return pl.pallas_call(
            _tc_body_first,
            grid=(QBLOCKS,),
            in_specs=in_specs,
            out_specs=out_spec,
            out_shape=jax.ShapeDtypeStruct((L, B, D), jnp.float32),
        )(e_q, W1, W2)
    return pl.pallas_call(
        _tc_body_next,
        grid=(QBLOCKS,),
        in_specs=in_specs + [pl.BlockSpec(memory_space=pl.ANY)],
        out_specs=out_spec,
        out_shape=jax.ShapeDtypeStruct((L, B, D), jnp.float32),
        input_output_aliases={3: 0},
    )(e_q, W1, W2, o_prev)


def kernel(x, emb_table, W1, W2):
    xt = jnp.transpose(x.astype(jnp.int32), (1, 0))
    es = [_sc_gather(emb_table, xt, q) for q in range(NQ)]
    out = None
    for q in range(NQ):
        out = _tc_stripe(es[q], W1, W2, q, out)
    return jnp.transpose(out, (1, 0, 2))
